# Initial kernel scaffold; baseline (speedup 1.0000x reference)
import jax
import jax.numpy as jnp
from jax.experimental import pallas as pl


def kernel(query, value, reference_points, spatial_shapes, level_start_index, Wv, bv, Woff, boff, Wa, ba, Wo, bo):
    raise NotImplementedError("write your pallas kernel here")


# speedup vs baseline: 19.3754x; 19.3754x over previous
import functools

import jax
import jax.numpy as jnp
import numpy as np
from jax import lax
from jax.experimental import pallas as pl
from jax.experimental.pallas import tpu as pltpu
from jax.experimental.pallas import tpu_sc as plsc

NH, NL, NP = 8, 4, 4
BS, NQ, D = 2, 900, 256
HD = D // NH
NV = 21760
NBQ = BS * NQ
LEVELS = (128, 64, 32, 16)
OFFS = (0, 16384, 20480, 21504)
ROWS_PER_W = 64
NW = 32

_PERM = np.empty(256, dtype=np.int32)
for _h in range(NH):
    for _c in range(2):
        for _s in range(16):
            _PERM[_h * 32 + _c * 16 + _s] = _h * 32 + (_s // 4) * 8 + (_s % 4) * 2 + _c


def _matmul_bias_body(x_ref, w_ref, b_ref, o_ref):
    o_ref[0] = jnp.dot(x_ref[0], w_ref[...],
                       preferred_element_type=jnp.float32) + b_ref[...]


def _proj_v(value, wvt, bv2):
    blk = 1280
    grid = (BS, NV // blk)
    return pl.pallas_call(
        _matmul_bias_body,
        grid=grid,
        in_specs=[
            pl.BlockSpec((1, blk, D), lambda b, i: (b, i, 0)),
            pl.BlockSpec((D, D), lambda b, i: (0, 0)),
            pl.BlockSpec((1, D), lambda b, i: (0, 0)),
        ],
        out_specs=pl.BlockSpec((1, blk, D), lambda b, i: (b, i, 0)),
        out_shape=jax.ShapeDtypeStruct((BS, NV, D), jnp.float32),
    )(value, wvt, bv2)


def _params_body(q_ref, woff_ref, boff_ref, wa_ref, ba_ref, off_ref, alog_ref):
    q = q_ref[0]
    off_ref[0] = jnp.dot(q, woff_ref[...],
                         preferred_element_type=jnp.float32) + boff_ref[...]
    alog_ref[0] = jnp.dot(q, wa_ref[...],
                          preferred_element_type=jnp.float32) + ba_ref[...]


def _params(query, wofft, boff2, wat, ba2):
    return pl.pallas_call(
        _params_body,
        grid=(BS,),
        in_specs=[
            pl.BlockSpec((1, NQ, D), lambda b: (b, 0, 0)),
            pl.BlockSpec((D, D), lambda b: (0, 0)),
            pl.BlockSpec((1, D), lambda b: (0, 0)),
            pl.BlockSpec((D, 128), lambda b: (0, 0)),
            pl.BlockSpec((1, 128), lambda b: (0, 0)),
        ],
        out_specs=[
            pl.BlockSpec((1, NQ, D), lambda b: (b, 0, 0)),
            pl.BlockSpec((1, NQ, 128), lambda b: (b, 0, 0)),
        ],
        out_shape=[
            jax.ShapeDtypeStruct((BS, NQ, D), jnp.float32),
            jax.ShapeDtypeStruct((BS, NQ, 128), jnp.float32),
        ],
    )(query, wofft, boff2, wat, ba2)


def _out_proj(msda, wot, bo2):
    return pl.pallas_call(
        _matmul_bias_body,
        grid=(BS,),
        in_specs=[
            pl.BlockSpec((1, NQ, D), lambda b: (b, 0, 0)),
            pl.BlockSpec((D, D), lambda b: (0, 0)),
            pl.BlockSpec((1, D), lambda b: (0, 0)),
        ],
        out_specs=pl.BlockSpec((1, NQ, D), lambda b: (b, 0, 0)),
        out_shape=jax.ShapeDtypeStruct((BS, NQ, D), jnp.float32),
    )(msda, wot, bo2)


def _sc_body(off_hbm, alog_hbm, ref_hbm, vp_hbm, out_hbm,
             offbuf, alogbuf, refbuf, idxbuf, wtbuf, rowsbuf, outbuf, sem):
    wid = lax.axis_index("s") * 2 + lax.axis_index("c")
    s0 = wid * 56

    pltpu.sync_copy(off_hbm.at[pl.ds(s0, ROWS_PER_W)], offbuf)
    pltpu.sync_copy(alog_hbm.at[pl.ds(s0, ROWS_PER_W)], alogbuf)
    pltpu.sync_copy(ref_hbm.at[pl.ds(s0 * 8, ROWS_PER_W * 8)],
                    refbuf.at[pl.ds(0, ROWS_PER_W * 8)])

    lane = lax.iota(jnp.int32, 16)
    lvl = lane >> 2
    w16i = jnp.int32(128) >> lvl
    w16f = w16i.astype(jnp.float32)
    wm1 = w16i - 1
    offs16 = jnp.where(lvl == 0, 0,
                       jnp.where(lvl == 1, 16384,
                                 jnp.where(lvl == 2, 20480, 21504)))
    c2l = lvl * 2
    zero16 = jnp.zeros((16,), jnp.float32)

    def _vgather(vec, idx):
        return lax.gather(
            vec, idx.reshape(16, 1),
            lax.GatherDimensionNumbers(
                offset_dims=(), collapsed_slice_dims=(0,),
                start_index_map=(0,)),
            (1,), mode=lax.GatherScatterMode.PROMISE_IN_BOUNDS)

    def bq_body(i, carry):
        bq = s0 + i
        base_b = (bq // NQ) * (NV * NH)
        rv16 = refbuf[pl.ds(i * 8, 16)]
        refx = _vgather(rv16, c2l)
        refy = _vgather(rv16, c2l + 1)
        refxw = refx * w16f - 0.5
        refyw = refy * w16f - 0.5

        for h in range(NH):
            vx = offbuf[i, pl.ds(h * 32, 16)]
            vy = offbuf[i, pl.ds(h * 32 + 16, 16)]
            px = refxw + vx
            py = refyw + vy
            xt = px.astype(jnp.int32)
            xtf = xt.astype(jnp.float32)
            x0 = jnp.where(xtf > px, xt - 1, xt)
            lx = px - x0.astype(jnp.float32)
            yt = py.astype(jnp.int32)
            ytf = yt.astype(jnp.float32)
            y0 = jnp.where(ytf > py, yt - 1, yt)
            ly = py - y0.astype(jnp.float32)
            vx0 = (x0 >= 0) & (x0 < w16i)
            vx1 = (x0 >= -1) & (x0 < wm1)
            vy0 = (y0 >= 0) & (y0 < w16i)
            vy1 = (y0 >= -1) & (y0 < wm1)
            omlx = 1.0 - lx
            omly = 1.0 - ly
            bwx0 = jnp.where(vx0, omlx, zero16)
            bwx1 = jnp.where(vx1, lx, zero16)
            bwy0 = jnp.where(vy0, omly, zero16)
            bwy1 = jnp.where(vy1, ly, zero16)
            a16 = alogbuf[i, pl.ds(h * 16, 16)]
            m = a16
            for off in (8, 4, 2, 1):
                m = jnp.maximum(m, _vgather(m, lane ^ off))
            e = jnp.exp(a16 - m)
            s = e
            for off in (8, 4, 2, 1):
                s = s + _vgather(s, lane ^ off)
            at = e / s
            ax0 = bwx0 * at
            ax1 = bwx1 * at
            w00 = ax0 * bwy0
            w01 = ax1 * bwy0
            w10 = ax0 * bwy1
            w11 = ax1 * bwy1
            xc0 = jnp.minimum(jnp.maximum(x0, 0), wm1)
            xc1 = jnp.minimum(jnp.maximum(x0 + 1, 0), wm1)
            yc0 = jnp.minimum(jnp.maximum(y0, 0), wm1)
            yc1 = jnp.minimum(jnp.maximum(y0 + 1, 0), wm1)
            py0 = offs16 + yc0 * w16i
            py1 = offs16 + yc1 * w16i
            base = base_b + h
            i00 = (py0 + xc0) * 8 + base
            i01 = (py0 + xc1) * 8 + base
            i10 = (py1 + xc0) * 8 + base
            i11 = (py1 + xc1) * 8 + base
            row = h // 2
            col = (h % 2) * 64
            idxbuf[row, pl.ds(col, 16)] = i00
            idxbuf[row, pl.ds(col + 16, 16)] = i01
            idxbuf[row, pl.ds(col + 32, 16)] = i10
            idxbuf[row, pl.ds(col + 48, 16)] = i11
            wtbuf[h, pl.ds(0, 16)] = w00
            wtbuf[h, pl.ds(16, 16)] = w01
            wtbuf[h, pl.ds(32, 16)] = w10
            wtbuf[h, pl.ds(48, 16)] = w11

        cps = [pltpu.async_copy(vp_hbm.at[idxbuf.at[c]],
                                rowsbuf.at[pl.ds(c * 128, 128)], sem)
               for c in range(4)]
        for cp in cps:
            cp.wait()

        for h in range(NH):
            wv = [wtbuf[h, pl.ds(c * 16, 16)] for c in range(4)]

            def jbody(j, acc, h=h, wv=wv):
                acc0, acc1 = acc
                jf = jnp.full((16,), j, jnp.int32)
                for c in range(4):
                    wk = _vgather(wv[c], jf)
                    k = h * 64 + c * 16 + j
                    acc0 = acc0 + wk * rowsbuf[k, pl.ds(0, 16)]
                    acc1 = acc1 + wk * rowsbuf[k, pl.ds(16, 16)]
                return (acc0, acc1)

            acc0, acc1 = lax.fori_loop(0, 16, jbody, (zero16, zero16))
            outbuf[i, pl.ds(h * 32, 16)] = acc0
            outbuf[i, pl.ds(h * 32 + 16, 16)] = acc1
        return carry

    lax.fori_loop(0, ROWS_PER_W, bq_body, 0)
    pltpu.sync_copy(outbuf, out_hbm.at[pl.ds(s0, ROWS_PER_W)])


def _sc_msda(offv, alogv, refv, vp2):
    mesh = plsc.VectorSubcoreMesh(core_axis_name="c", subcore_axis_name="s")
    f = functools.partial(
        pl.kernel,
        mesh=mesh,
        compiler_params=pltpu.CompilerParams(use_tc_tiling_on_sc=False),
        out_type=jax.ShapeDtypeStruct((NBQ, D), jnp.float32),
        scratch_types=[
            pltpu.VMEM((ROWS_PER_W, D), jnp.float32),
            pltpu.VMEM((ROWS_PER_W, 128), jnp.float32),
            pltpu.VMEM((ROWS_PER_W * 8 + 8,), jnp.float32),
            pltpu.VMEM((4, 128), jnp.int32),
            pltpu.VMEM((8, 64), jnp.float32),
            pltpu.VMEM((512, 32), jnp.float32),
            pltpu.VMEM((ROWS_PER_W, D), jnp.float32),
            pltpu.SemaphoreType.DMA,
        ],
    )(_sc_body)
    return f(offv, alogv, refv, vp2)


def kernel(query, value, reference_points, spatial_shapes, level_start_index,
           Wv, bv, Woff, boff, Wa, ba, Wo, bo):
    del spatial_shapes, level_start_index
    perm = jnp.asarray(_PERM)
    wofft = Woff.T[:, perm]
    boff2 = boff[perm].reshape(1, D)
    vp = _proj_v(value, Wv.T, bv.reshape(1, D))
    off, alog = _params(query, wofft, boff2, Wa.T, ba.reshape(1, 128))
    offv = off.reshape(NBQ, D)
    alogv = alog.reshape(NBQ, 128)
    refv = jnp.concatenate(
        [reference_points.reshape(NBQ * NL * 2),
         jnp.zeros((8,), jnp.float32)])
    vp2 = vp.reshape(BS * NV * NH, HD)
    msda = _sc_msda(offv, alogv, refv, vp2)
    return _out_proj(msda.reshape(BS, NQ, D), Wo.T, bo.reshape(1, D))


# speedup vs baseline: 22.4480x; 1.1586x over previous
import functools

import jax
import jax.numpy as jnp
import numpy as np
from jax import lax
from jax.experimental import pallas as pl
from jax.experimental.pallas import tpu as pltpu
from jax.experimental.pallas import tpu_sc as plsc

NH, NL, NP = 8, 4, 4
BS, NQ, D = 2, 900, 256
HD = D // NH
NV = 21760
NBQ = BS * NQ
LEVELS = (128, 64, 32, 16)
OFFS = (0, 16384, 20480, 21504)
ROWS_PER_W = 64
NW = 32

_PERM = np.empty(256, dtype=np.int32)
for _h in range(NH):
    for _c in range(2):
        for _s in range(16):
            _PERM[_h * 32 + _c * 16 + _s] = _h * 32 + (_s // 4) * 8 + (_s % 4) * 2 + _c


def _matmul_bias_body(x_ref, w_ref, b_ref, o_ref):
    o_ref[0] = jnp.dot(x_ref[0], w_ref[...],
                       preferred_element_type=jnp.float32) + b_ref[...]


def _proj_v_body(x_ref, w_ref, b_ref, o_ref):
    y = jnp.dot(x_ref[0], w_ref[...],
                preferred_element_type=jnp.float32) + b_ref[...]
    o_ref[...] = y.reshape(o_ref.shape)


def _proj_v(value, wvt, bv2):
    blk = 1280
    grid = (BS, NV // blk)
    return pl.pallas_call(
        _proj_v_body,
        grid=grid,
        in_specs=[
            pl.BlockSpec((1, blk, D), lambda b, i: (b, i, 0)),
            pl.BlockSpec((D, D), lambda b, i: (0, 0)),
            pl.BlockSpec((1, D), lambda b, i: (0, 0)),
        ],
        out_specs=pl.BlockSpec((blk * 2, 128), lambda b, i: (b * 17 + i, 0)),
        out_shape=jax.ShapeDtypeStruct((BS * NV * 2, 128), jnp.float32),
    )(value, wvt, bv2)


def _params_body(q_ref, woff_ref, boff_ref, wa_ref, ba_ref, off_ref, alog_ref):
    q = q_ref[0]
    off_ref[0] = jnp.dot(q, woff_ref[...],
                         preferred_element_type=jnp.float32) + boff_ref[...]
    alog_ref[0] = jnp.dot(q, wa_ref[...],
                          preferred_element_type=jnp.float32) + ba_ref[...]


def _params(query, wofft, boff2, wat, ba2):
    return pl.pallas_call(
        _params_body,
        grid=(BS,),
        in_specs=[
            pl.BlockSpec((1, NQ, D), lambda b: (b, 0, 0)),
            pl.BlockSpec((D, D), lambda b: (0, 0)),
            pl.BlockSpec((1, D), lambda b: (0, 0)),
            pl.BlockSpec((D, 128), lambda b: (0, 0)),
            pl.BlockSpec((1, 128), lambda b: (0, 0)),
        ],
        out_specs=[
            pl.BlockSpec((1, NQ, D), lambda b: (b, 0, 0)),
            pl.BlockSpec((1, NQ, 128), lambda b: (b, 0, 0)),
        ],
        out_shape=[
            jax.ShapeDtypeStruct((BS, NQ, D), jnp.float32),
            jax.ShapeDtypeStruct((BS, NQ, 128), jnp.float32),
        ],
    )(query, wofft, boff2, wat, ba2)


def _out_proj(msda, wot, bo2):
    return pl.pallas_call(
        _matmul_bias_body,
        grid=(BS,),
        in_specs=[
            pl.BlockSpec((1, NQ, D), lambda b: (b, 0, 0)),
            pl.BlockSpec((D, D), lambda b: (0, 0)),
            pl.BlockSpec((1, D), lambda b: (0, 0)),
        ],
        out_specs=pl.BlockSpec((1, NQ, D), lambda b: (b, 0, 0)),
        out_shape=jax.ShapeDtypeStruct((BS, NQ, D), jnp.float32),
    )(msda, wot, bo2)


def _sc_body(off_hbm, alog_hbm, ref_hbm, vp_hbm, out_hbm,
             offbuf, alogbuf, refbuf, idx_a, idx_b, wt_a, wt_b,
             rows_a, rows_b, outbuf, sem_a, sem_b):
    wid = lax.axis_index("s") * 2 + lax.axis_index("c")
    s0 = wid * 56

    pltpu.sync_copy(off_hbm.at[pl.ds(s0, ROWS_PER_W)], offbuf)
    pltpu.sync_copy(alog_hbm.at[pl.ds(s0, ROWS_PER_W)], alogbuf)
    pltpu.sync_copy(ref_hbm.at[pl.ds(s0 * 8, ROWS_PER_W * 8)],
                    refbuf.at[pl.ds(0, ROWS_PER_W * 8)])

    lane = lax.iota(jnp.int32, 16)
    lvl = lane >> 2
    w16i = jnp.int32(128) >> lvl
    w16f = w16i.astype(jnp.float32)
    wm1 = w16i - 1
    offs16 = jnp.where(lvl == 0, 0,
                       jnp.where(lvl == 1, 16384,
                                 jnp.where(lvl == 2, 20480, 21504)))
    c2l = lvl * 2
    zero16 = jnp.zeros((16,), jnp.float32)

    def _vgather(vec, idx):
        return lax.gather(
            vec, idx.reshape(16, 1),
            lax.GatherDimensionNumbers(
                offset_dims=(), collapsed_slice_dims=(0,),
                start_index_map=(0,)),
            (1,), mode=lax.GatherScatterMode.PROMISE_IN_BOUNDS)

    def build(i, idxbuf, wtbuf):
        bq = s0 + i
        base_b = (bq // NQ) * (NV * NH)
        rv16 = refbuf[pl.ds(i * 8, 16)]
        refx = _vgather(rv16, c2l)
        refy = _vgather(rv16, c2l + 1)
        refxw = refx * w16f - 0.5
        refyw = refy * w16f - 0.5

        for h in range(NH):
            vx = offbuf[i, pl.ds(h * 32, 16)]
            vy = offbuf[i, pl.ds(h * 32 + 16, 16)]
            px = refxw + vx
            py = refyw + vy
            xt = px.astype(jnp.int32)
            xtf = xt.astype(jnp.float32)
            x0 = jnp.where(xtf > px, xt - 1, xt)
            lx = px - x0.astype(jnp.float32)
            yt = py.astype(jnp.int32)
            ytf = yt.astype(jnp.float32)
            y0 = jnp.where(ytf > py, yt - 1, yt)
            ly = py - y0.astype(jnp.float32)
            vx0 = (x0 >= 0) & (x0 < w16i)
            vx1 = (x0 >= -1) & (x0 < wm1)
            vy0 = (y0 >= 0) & (y0 < w16i)
            vy1 = (y0 >= -1) & (y0 < wm1)
            omlx = 1.0 - lx
            omly = 1.0 - ly
            bwx0 = jnp.where(vx0, omlx, zero16)
            bwx1 = jnp.where(vx1, lx, zero16)
            bwy0 = jnp.where(vy0, omly, zero16)
            bwy1 = jnp.where(vy1, ly, zero16)
            a16 = alogbuf[i, pl.ds(h * 16, 16)]
            m = a16
            for off in (8, 4, 2, 1):
                m = jnp.maximum(m, _vgather(m, lane ^ off))
            e = jnp.exp(a16 - m)
            s = e
            for off in (8, 4, 2, 1):
                s = s + _vgather(s, lane ^ off)
            at = e / s
            ax0 = bwx0 * at
            ax1 = bwx1 * at
            w00 = ax0 * bwy0
            w01 = ax1 * bwy0
            w10 = ax0 * bwy1
            w11 = ax1 * bwy1
            xc0 = jnp.minimum(jnp.maximum(x0, 0), wm1)
            xc1 = jnp.minimum(jnp.maximum(x0 + 1, 0), wm1)
            yc0 = jnp.minimum(jnp.maximum(y0, 0), wm1)
            yc1 = jnp.minimum(jnp.maximum(y0 + 1, 0), wm1)
            py0 = offs16 + yc0 * w16i
            py1 = offs16 + yc1 * w16i
            base = base_b + h
            i00 = (py0 + xc0) * 8 + base
            i01 = (py0 + xc1) * 8 + base
            i10 = (py1 + xc0) * 8 + base
            i11 = (py1 + xc1) * 8 + base
            row = h // 2
            col = (h % 2) * 64
            idxbuf[row, pl.ds(col, 16)] = i00
            idxbuf[row, pl.ds(col + 16, 16)] = i01
            idxbuf[row, pl.ds(col + 32, 16)] = i10
            idxbuf[row, pl.ds(col + 48, 16)] = i11
            wtbuf[h, pl.ds(0, 16)] = w00
            wtbuf[h, pl.ds(16, 16)] = w01
            wtbuf[h, pl.ds(32, 16)] = w10
            wtbuf[h, pl.ds(48, 16)] = w11

    def _gcps(idxbuf, rowsbuf, sem):
        return [pltpu.make_async_copy(vp_hbm.at[idxbuf.at[c]],
                                      rowsbuf.at[pl.ds(c * 128, 128)], sem)
                for c in range(4)]

    def fire(idxbuf, rowsbuf, sem):
        for cp in _gcps(idxbuf, rowsbuf, sem):
            cp.start()

    def drain(idxbuf, rowsbuf, sem):
        for cp in _gcps(idxbuf, rowsbuf, sem):
            cp.wait()

    def accum(i, wtbuf, rowsbuf):
        for h in range(NH):
            wv = [wtbuf[h, pl.ds(c * 16, 16)] for c in range(4)]

            def jbody(j4, acc, h=h, wv=wv):
                acc0, acc1 = acc
                for jj in range(4):
                    j = j4 * 4 + jj
                    jf = jnp.full((16,), j, jnp.int32)
                    for c in range(4):
                        wk = _vgather(wv[c], jf)
                        k = h * 64 + c * 16 + j
                        acc0 = acc0 + wk * rowsbuf[k, pl.ds(0, 16)]
                        acc1 = acc1 + wk * rowsbuf[k, pl.ds(16, 16)]
                return (acc0, acc1)

            acc0, acc1 = lax.fori_loop(0, 4, jbody, (zero16, zero16))
            outbuf[i, pl.ds(h * 32, 16)] = acc0
            outbuf[i, pl.ds(h * 32 + 16, 16)] = acc1

    build(0, idx_a, wt_a)
    fire(idx_a, rows_a, sem_a)

    def pair(k, carry):
        i0 = 2 * k
        build(i0 + 1, idx_b, wt_b)
        fire(idx_b, rows_b, sem_b)
        drain(idx_a, rows_a, sem_a)
        accum(i0, wt_a, rows_a)

        @pl.when(k < ROWS_PER_W // 2 - 1)
        def _():
            build(i0 + 2, idx_a, wt_a)
            fire(idx_a, rows_a, sem_a)

        drain(idx_b, rows_b, sem_b)
        accum(i0 + 1, wt_b, rows_b)
        return carry

    lax.fori_loop(0, ROWS_PER_W // 2, pair, 0)
    pltpu.sync_copy(outbuf, out_hbm.at[pl.ds(s0, ROWS_PER_W)])


def _sc_msda(offv, alogv, refv, vp2):
    mesh = plsc.VectorSubcoreMesh(core_axis_name="c", subcore_axis_name="s")
    f = functools.partial(
        pl.kernel,
        mesh=mesh,
        compiler_params=pltpu.CompilerParams(use_tc_tiling_on_sc=False),
        out_type=jax.ShapeDtypeStruct((NBQ, D), jnp.float32),
        scratch_types=[
            pltpu.VMEM((ROWS_PER_W, D), jnp.float32),
            pltpu.VMEM((ROWS_PER_W, 128), jnp.float32),
            pltpu.VMEM((ROWS_PER_W * 8 + 8,), jnp.float32),
            pltpu.VMEM((4, 128), jnp.int32),
            pltpu.VMEM((4, 128), jnp.int32),
            pltpu.VMEM((8, 64), jnp.float32),
            pltpu.VMEM((8, 64), jnp.float32),
            pltpu.VMEM((512, 32), jnp.float32),
            pltpu.VMEM((512, 32), jnp.float32),
            pltpu.VMEM((ROWS_PER_W, D), jnp.float32),
            pltpu.SemaphoreType.DMA,
            pltpu.SemaphoreType.DMA,
        ],
    )(_sc_body)
    return f(offv, alogv, refv, vp2)


def kernel(query, value, reference_points, spatial_shapes, level_start_index,
           Wv, bv, Woff, boff, Wa, ba, Wo, bo):
    del spatial_shapes, level_start_index
    perm = jnp.asarray(_PERM)
    wofft = Woff.T[:, perm]
    boff2 = boff[perm].reshape(1, D)
    vp = _proj_v(value, Wv.T, bv.reshape(1, D))
    off, alog = _params(query, wofft, boff2, Wa.T, ba.reshape(1, 128))
    offv = off.reshape(NBQ, D)
    alogv = alog.reshape(NBQ, 128)
    refv = jnp.concatenate(
        [reference_points.reshape(NBQ * NL * 2),
         jnp.zeros((8,), jnp.float32)])
    vp2 = vp.reshape(BS * NV * NH, HD)
    msda = _sc_msda(offv, alogv, refv, vp2)
    return _out_proj(msda.reshape(BS, NQ, D), Wo.T, bo.reshape(1, D))


# speedup vs baseline: 30.8130x; 1.3726x over previous
import functools

import jax
import jax.numpy as jnp
import numpy as np
from jax import lax
from jax.experimental import pallas as pl
from jax.experimental.pallas import tpu as pltpu
from jax.experimental.pallas import tpu_sc as plsc

NH, NL, NP = 8, 4, 4
BS, NQ, D = 2, 900, 256
HD = D // NH
NV = 21760
NBQ = BS * NQ
LEVELS = (128, 64, 32, 16)
OFFS = (0, 16384, 20480, 21504)
ROWS_PER_W = 64
NW = 32

_LOPERM = np.array([h * 32 + k for h in range(NH) for k in range(16)],
                   dtype=np.int32)

_PERM = np.empty(256, dtype=np.int32)
for _h in range(NH):
    for _c in range(2):
        for _s in range(16):
            _PERM[_h * 32 + _c * 16 + _s] = _h * 32 + (_s // 4) * 8 + (_s % 4) * 2 + _c


def _matmul_bias_body(x_ref, w_ref, b_ref, o_ref):
    o_ref[0] = jnp.dot(x_ref[0], w_ref[...],
                       preferred_element_type=jnp.float32) + b_ref[...]


def _rn_bf16_bits(y):
    u = lax.bitcast_convert_type(y, jnp.uint32)
    return (u + jnp.uint32(0x7FFF) + ((u >> 16) & jnp.uint32(1))) >> 16


def _proj_v_body(x_ref, wlo_ref, whi_ref, blo_ref, bhi_ref, o_ref):
    x = x_ref[0]
    ylo = jnp.dot(x, wlo_ref[...],
                  preferred_element_type=jnp.float32) + blo_ref[...]
    yhi = jnp.dot(x, whi_ref[...],
                  preferred_element_type=jnp.float32) + bhi_ref[...]
    o_ref[...] = _rn_bf16_bits(ylo) | (_rn_bf16_bits(yhi) << 16)


def _proj_params_body(x_ref, wlo_ref, whi_ref, blo_ref, bhi_ref,
                      q_ref, woff_ref, boff_ref, wa_ref, ba_ref,
                      o_ref, off_ref, alog_ref):
    _proj_v_body(x_ref, wlo_ref, whi_ref, blo_ref, bhi_ref, o_ref)

    @pl.when(pl.program_id(1) == NV // 1280 - 1)
    def _():
        q = q_ref[0]
        off_ref[0] = jnp.dot(q, woff_ref[...],
                             preferred_element_type=jnp.float32) + boff_ref[...]
        alog_ref[0] = jnp.dot(q, wa_ref[...],
                              preferred_element_type=jnp.float32) + ba_ref[...]


def _proj_v(value, wlo, whi, blo, bhi, query, wofft, boff2, wat, ba2):
    blk = 1280
    grid = (BS, NV // blk)
    return pl.pallas_call(
        _proj_params_body,
        grid=grid,
        in_specs=[
            pl.BlockSpec((1, blk, D), lambda b, i: (b, i, 0)),
            pl.BlockSpec((D, 128), lambda b, i: (0, 0)),
            pl.BlockSpec((D, 128), lambda b, i: (0, 0)),
            pl.BlockSpec((1, 128), lambda b, i: (0, 0)),
            pl.BlockSpec((1, 128), lambda b, i: (0, 0)),
            pl.BlockSpec((1, NQ, D), lambda b, i: (b, 0, 0)),
            pl.BlockSpec((D, D), lambda b, i: (0, 0)),
            pl.BlockSpec((1, D), lambda b, i: (0, 0)),
            pl.BlockSpec((D, 128), lambda b, i: (0, 0)),
            pl.BlockSpec((1, 128), lambda b, i: (0, 0)),
        ],
        out_specs=[
            pl.BlockSpec((blk, 128), lambda b, i: (b * 17 + i, 0)),
            pl.BlockSpec((1, NQ, D), lambda b, i: (b, 0, 0)),
            pl.BlockSpec((1, NQ, 128), lambda b, i: (b, 0, 0)),
        ],
        out_shape=[
            jax.ShapeDtypeStruct((BS * NV, 128), jnp.uint32),
            jax.ShapeDtypeStruct((BS, NQ, D), jnp.float32),
            jax.ShapeDtypeStruct((BS, NQ, 128), jnp.float32),
        ],
    )(value, wlo, whi, blo, bhi, query, wofft, boff2, wat, ba2)


def _params_body(q_ref, woff_ref, boff_ref, wa_ref, ba_ref, off_ref, alog_ref):
    q = q_ref[0]
    off_ref[0] = jnp.dot(q, woff_ref[...],
                         preferred_element_type=jnp.float32) + boff_ref[...]
    alog_ref[0] = jnp.dot(q, wa_ref[...],
                          preferred_element_type=jnp.float32) + ba_ref[...]


def _params(query, wofft, boff2, wat, ba2):
    return pl.pallas_call(
        _params_body,
        grid=(BS,),
        in_specs=[
            pl.BlockSpec((1, NQ, D), lambda b: (b, 0, 0)),
            pl.BlockSpec((D, D), lambda b: (0, 0)),
            pl.BlockSpec((1, D), lambda b: (0, 0)),
            pl.BlockSpec((D, 128), lambda b: (0, 0)),
            pl.BlockSpec((1, 128), lambda b: (0, 0)),
        ],
        out_specs=[
            pl.BlockSpec((1, NQ, D), lambda b: (b, 0, 0)),
            pl.BlockSpec((1, NQ, 128), lambda b: (b, 0, 0)),
        ],
        out_shape=[
            jax.ShapeDtypeStruct((BS, NQ, D), jnp.float32),
            jax.ShapeDtypeStruct((BS, NQ, 128), jnp.float32),
        ],
    )(query, wofft, boff2, wat, ba2)


def _out_proj(msda, wot, bo2):
    return pl.pallas_call(
        _matmul_bias_body,
        grid=(BS,),
        in_specs=[
            pl.BlockSpec((1, NQ, D), lambda b: (b, 0, 0)),
            pl.BlockSpec((D, D), lambda b: (0, 0)),
            pl.BlockSpec((1, D), lambda b: (0, 0)),
        ],
        out_specs=pl.BlockSpec((1, NQ, D), lambda b: (b, 0, 0)),
        out_shape=jax.ShapeDtypeStruct((BS, NQ, D), jnp.float32),
    )(msda, wot, bo2)


def _sc_body(off_hbm, alog_hbm, ref_hbm, vp_hbm, out_hbm,
             offbuf, alogbuf, refbuf, idx_a, idx_b, wt_a, wt_b,
             rows_a, rows_b, outbuf, sem_a, sem_b):
    wid = lax.axis_index("s") * 2 + lax.axis_index("c")
    s0 = wid * 56

    pltpu.sync_copy(off_hbm.at[pl.ds(s0, ROWS_PER_W)], offbuf)
    pltpu.sync_copy(alog_hbm.at[pl.ds(s0, ROWS_PER_W)], alogbuf)
    pltpu.sync_copy(ref_hbm.at[pl.ds(s0 * 8, ROWS_PER_W * 8)],
                    refbuf.at[pl.ds(0, ROWS_PER_W * 8)])

    lane = lax.iota(jnp.int32, 16)
    lvl = lane >> 2
    w16i = jnp.int32(128) >> lvl
    w16f = w16i.astype(jnp.float32)
    wm1 = w16i - 1
    offs16 = jnp.where(lvl == 0, 0,
                       jnp.where(lvl == 1, 16384,
                                 jnp.where(lvl == 2, 20480, 21504)))
    c2l = lvl * 2
    zero16 = jnp.zeros((16,), jnp.float32)

    def _vgather(vec, idx):
        return lax.gather(
            vec, idx.reshape(16, 1),
            lax.GatherDimensionNumbers(
                offset_dims=(), collapsed_slice_dims=(0,),
                start_index_map=(0,)),
            (1,), mode=lax.GatherScatterMode.PROMISE_IN_BOUNDS)

    def build(i, idxbuf, wtbuf):
        bq = s0 + i
        base_b = jnp.where(bq >= NQ, NV * NH, 0)
        rv16 = refbuf[pl.ds(i * 8, 16)]
        refx = _vgather(rv16, c2l)
        refy = _vgather(rv16, c2l + 1)
        refxw = refx * w16f - 0.5
        refyw = refy * w16f - 0.5

        for h in range(NH):
            vx = offbuf[i, pl.ds(h * 32, 16)]
            vy = offbuf[i, pl.ds(h * 32 + 16, 16)]
            px = refxw + vx
            py = refyw + vy
            xt = px.astype(jnp.int32)
            xtf = xt.astype(jnp.float32)
            x0 = jnp.where(xtf > px, xt - 1, xt)
            lx = px - x0.astype(jnp.float32)
            yt = py.astype(jnp.int32)
            ytf = yt.astype(jnp.float32)
            y0 = jnp.where(ytf > py, yt - 1, yt)
            ly = py - y0.astype(jnp.float32)
            vx0 = (x0 >= 0) & (x0 < w16i)
            vx1 = (x0 >= -1) & (x0 < wm1)
            vy0 = (y0 >= 0) & (y0 < w16i)
            vy1 = (y0 >= -1) & (y0 < wm1)
            omlx = 1.0 - lx
            omly = 1.0 - ly
            bwx0 = jnp.where(vx0, omlx, zero16)
            bwx1 = jnp.where(vx1, lx, zero16)
            bwy0 = jnp.where(vy0, omly, zero16)
            bwy1 = jnp.where(vy1, ly, zero16)
            a16 = alogbuf[i, pl.ds(h * 16, 16)]
            m = a16
            for off in (8, 4, 2, 1):
                m = jnp.maximum(m, _vgather(m, lane ^ off))
            e = jnp.exp(a16 - m)
            s = e
            for off in (8, 4, 2, 1):
                s = s + _vgather(s, lane ^ off)
            at = e / s
            ax0 = bwx0 * at
            ax1 = bwx1 * at
            w00 = ax0 * bwy0
            w01 = ax1 * bwy0
            w10 = ax0 * bwy1
            w11 = ax1 * bwy1
            xc0 = jnp.minimum(jnp.maximum(x0, 0), wm1)
            xc1 = jnp.minimum(jnp.maximum(x0 + 1, 0), wm1)
            yc0 = jnp.minimum(jnp.maximum(y0, 0), wm1)
            yc1 = jnp.minimum(jnp.maximum(y0 + 1, 0), wm1)
            py0 = offs16 + yc0 * w16i
            py1 = offs16 + yc1 * w16i
            base = base_b + h
            i00 = (py0 + xc0) * 8 + base
            i01 = (py0 + xc1) * 8 + base
            i10 = (py1 + xc0) * 8 + base
            i11 = (py1 + xc1) * 8 + base
            row = h // 2
            col = (h % 2) * 64
            idxbuf[row, pl.ds(col, 16)] = i00
            idxbuf[row, pl.ds(col + 16, 16)] = i01
            idxbuf[row, pl.ds(col + 32, 16)] = i10
            idxbuf[row, pl.ds(col + 48, 16)] = i11
            wtbuf[h, pl.ds(0, 16)] = w00
            wtbuf[h, pl.ds(16, 16)] = w01
            wtbuf[h, pl.ds(32, 16)] = w10
            wtbuf[h, pl.ds(48, 16)] = w11

    def _gcps(idxbuf, rowsbuf, sem):
        return [pltpu.make_async_copy(vp_hbm.at[idxbuf.at[c]],
                                      rowsbuf.at[pl.ds(c * 128, 128)], sem)
                for c in range(4)]

    def fire(idxbuf, rowsbuf, sem):
        for cp in _gcps(idxbuf, rowsbuf, sem):
            cp.start()

    def drain(idxbuf, rowsbuf, sem):
        for cp in _gcps(idxbuf, rowsbuf, sem):
            cp.wait()

    def accum(i, wtbuf, rowsbuf):
        def hbody(h, carry):
            wv = [wtbuf[h, pl.ds(c * 16, 16)] for c in range(4)]
            base = h * 64
            acc = [zero16] * 8
            for j in range(16):
                jf = jnp.full((16,), j, jnp.int32)
                for c in range(4):
                    wk = _vgather(wv[c], jf)
                    w = rowsbuf[base + c * 16 + j]
                    ev = lax.bitcast_convert_type(w << 16, jnp.float32)
                    od = lax.bitcast_convert_type(w, jnp.float32)
                    acc[c * 2] = acc[c * 2] + wk * ev
                    acc[c * 2 + 1] = acc[c * 2 + 1] + wk * od
            orow = i * 8 + h
            outbuf[orow, pl.ds(0, 16)] = (acc[0] + acc[2]) + (acc[4] + acc[6])
            outbuf[orow, pl.ds(16, 16)] = (acc[1] + acc[3]) + (acc[5] + acc[7])
            return carry

        lax.fori_loop(0, NH, hbody, 0)

    build(0, idx_a, wt_a)
    fire(idx_a, rows_a, sem_a)

    def pair(k, carry):
        i0 = 2 * k
        build(i0 + 1, idx_b, wt_b)
        fire(idx_b, rows_b, sem_b)
        drain(idx_a, rows_a, sem_a)
        accum(i0, wt_a, rows_a)

        @pl.when(k < ROWS_PER_W // 2 - 1)
        def _():
            build(i0 + 2, idx_a, wt_a)
            fire(idx_a, rows_a, sem_a)

        drain(idx_b, rows_b, sem_b)
        accum(i0 + 1, wt_b, rows_b)
        return carry

    lax.fori_loop(0, ROWS_PER_W // 2, pair, 0)
    pltpu.sync_copy(outbuf, out_hbm.at[pl.ds(s0 * 8, ROWS_PER_W * 8)])


def _sc_msda(offv, alogv, refv, vp2):
    mesh = plsc.VectorSubcoreMesh(core_axis_name="c", subcore_axis_name="s")
    f = functools.partial(
        pl.kernel,
        mesh=mesh,
        compiler_params=pltpu.CompilerParams(use_tc_tiling_on_sc=False),
        out_type=jax.ShapeDtypeStruct((NBQ * NH, HD), jnp.float32),
        scratch_types=[
            pltpu.VMEM((ROWS_PER_W, D), jnp.float32),
            pltpu.VMEM((ROWS_PER_W, 128), jnp.float32),
            pltpu.VMEM((ROWS_PER_W * 8 + 8,), jnp.float32),
            pltpu.VMEM((4, 128), jnp.int32),
            pltpu.VMEM((4, 128), jnp.int32),
            pltpu.VMEM((8, 64), jnp.float32),
            pltpu.VMEM((8, 64), jnp.float32),
            pltpu.VMEM((512, 16), jnp.uint32),
            pltpu.VMEM((512, 16), jnp.uint32),
            pltpu.VMEM((ROWS_PER_W * 8, HD), jnp.float32),
            pltpu.SemaphoreType.DMA,
            pltpu.SemaphoreType.DMA,
        ],
    )(_sc_body)
    return f(offv, alogv, refv, vp2)


def kernel(query, value, reference_points, spatial_shapes, level_start_index,
           Wv, bv, Woff, boff, Wa, ba, Wo, bo):
    del spatial_shapes, level_start_index
    perm = jnp.asarray(_PERM)
    wofft = Woff.T[:, perm]
    boff2 = boff[perm].reshape(1, D)
    wvt = Wv.T
    lo = jnp.asarray(_LOPERM)
    hi = lo + 16
    vp, off, alog = _proj_v(value, wvt[:, lo], wvt[:, hi],
                            bv[lo].reshape(1, 128), bv[hi].reshape(1, 128),
                            query, wofft, boff2, Wa.T, ba.reshape(1, 128))
    offv = off.reshape(NBQ, D)
    alogv = alog.reshape(NBQ, 128)
    refv = reference_points.reshape(NBQ * NL * 2)
    vp2 = vp.reshape(BS * NV * NH, HD // 2)
    msda = _sc_msda(offv, alogv, refv, vp2)
    return _out_proj(msda.reshape(BS, NQ, D), Wo.T, bo.reshape(1, D))
